# Initial kernel scaffold; baseline (speedup 1.0000x reference)
import jax
import jax.numpy as jnp
from jax.experimental import pallas as pl


def kernel(x, sample_embedding, sample_bias, feature_embedding, feature_bias):
    raise NotImplementedError("write your pallas kernel here")


# speedup vs baseline: 1.2495x; 1.2495x over previous
import functools

import jax
import jax.numpy as jnp
from jax import lax
from jax.experimental import pallas as pl
from jax.experimental.pallas import tpu as pltpu
from jax.experimental.pallas import tpu_sc as plsc

BATCH = 16384
EMBED = 128
NC = 2
NS = 16
NW = NC * NS
BPW = BATCH // NW
CHUNK = 256
NCHUNK = BPW // CHUNK
L = 16
NVEC = EMBED // L


def _mf_body(si_hbm, fi_hbm, semb_hbm, femb_hbm, sb_hbm, fb_hbm, out_hbm,
             s_rows, f_rows, si_v, fi_v, sb_v, fb_v, out_v, sem):
    wid = lax.axis_index("s") * NC + lax.axis_index("c")
    base = wid * BPW

    pltpu.sync_copy(si_hbm.at[pl.ds(base, BPW)], si_v)
    pltpu.sync_copy(fi_hbm.at[pl.ds(base, BPW)], fi_v)
    pltpu.async_copy(sb_hbm.at[si_v], sb_v, sem).wait()
    pltpu.async_copy(fb_hbm.at[fi_v], fb_v, sem).wait()

    for c in range(NCHUNK):
        off = c * CHUNK
        pltpu.async_copy(semb_hbm.at[si_v.at[pl.ds(off, CHUNK)]], s_rows, sem).wait()
        pltpu.async_copy(femb_hbm.at[fi_v.at[pl.ds(off, CHUNK)]], f_rows, sem).wait()

        lane = lax.iota(jnp.int32, L)

        def grp_body(g, carry):
            r0 = g * L
            out16 = jnp.zeros((L,), jnp.float32)
            for j in range(L):
                r = r0 + j
                acc = s_rows[r, pl.ds(0, L)] * f_rows[r, pl.ds(0, L)]
                for k in range(1, NVEC):
                    acc = acc + s_rows[r, pl.ds(k * L, L)] * f_rows[r, pl.ds(k * L, L)]
                for sh in (8, 4, 2, 1):
                    acc = acc + acc.at[lane ^ sh].get(mode="promise_in_bounds")
                out16 = jnp.where(lane == j, acc, out16)
            o = off + r0
            out_v[pl.ds(o, L)] = out16 + sb_v[pl.ds(o, L)] + fb_v[pl.ds(o, L)]
            return carry

        lax.fori_loop(0, CHUNK // L, grp_body, 0)

    pltpu.sync_copy(out_v, out_hbm.at[pl.ds(base, BPW)])


@jax.jit
def _mf(si, fi, semb, femb, sb, fb):
    grid_kernel = functools.partial(
        pl.kernel,
        mesh=plsc.VectorSubcoreMesh(core_axis_name="c", subcore_axis_name="s"),
        out_type=jax.ShapeDtypeStruct((BATCH,), jnp.float32),
        scratch_types=[
            pltpu.VMEM((CHUNK, EMBED), jnp.float32),
            pltpu.VMEM((CHUNK, EMBED), jnp.float32),
            pltpu.VMEM((BPW,), jnp.int32),
            pltpu.VMEM((BPW,), jnp.int32),
            pltpu.VMEM((BPW,), jnp.float32),
            pltpu.VMEM((BPW,), jnp.float32),
            pltpu.VMEM((BPW,), jnp.float32),
            pltpu.SemaphoreType.DMA,
        ],
    )
    return grid_kernel(_mf_body)(si, fi, semb, femb, sb, fb)


def kernel(x, sample_embedding, sample_bias, feature_embedding, feature_bias):
    si = x[:, 0]
    fi = x[:, 1]
    return _mf(si, fi, sample_embedding, feature_embedding,
               sample_bias[:, 0], feature_bias[:, 0])
